# Initial kernel scaffold; baseline (speedup 1.0000x reference)
import jax
import jax.numpy as jnp
from jax.experimental import pallas as pl


def kernel(x_app_user, x_non_app_user, x_observation, x_target, ei_app, ei_non, ei_obs, Wrel_0_0, brel_0_0, Wroot_0_0, Wrel_0_1, brel_0_1, Wroot_0_1, Wrel_0_2, brel_0_2, Wroot_0_2, Wrel_1_0, brel_1_0, Wroot_1_0, Wrel_1_1, brel_1_1, Wroot_1_1, Wrel_1_2, brel_1_2, Wroot_1_2, mlp_W, mlp_b, lin_W, lin_b):
    raise NotImplementedError("write your pallas kernel here")


# speedup vs baseline: 3.0253x; 3.0253x over previous
import functools

import jax
import jax.numpy as jnp
from jax import lax
from jax.experimental import pallas as pl
from jax.experimental.pallas import tpu as pltpu
from jax.experimental.pallas import tpu_sc as plsc

N = 50000
D = 128
H = 128
OUT = 64
E = 200000

NSC = 2
NTILE = 16
LANES = 16
FCHUNK = D // LANES
FPC = FCHUNK // NSC
KCH = 256
NCH = 49
EPT = KCH * NCH
E_PAD = EPT * NTILE
NP = 50048
STRIPE = NP // NTILE
BN = 400


def _sc_body(xt0, src0, dst0, xt1, src1, dst1, xt2, src2, dst2, zeros,
             a00, a10, a01, a11, a02, a12,
             cidx_v, dst_v, rows_v, acc0_s, acc1_s, sem):
  c = lax.axis_index("c")
  tid = lax.axis_index("s")
  row0 = tid * STRIPE

  for xt, src, dst, a0, a1 in ((xt0, src0, dst0, a00, a10),
                               (xt1, src1, dst1, a01, a11),
                               (xt2, src2, dst2, a02, a12)):
    pltpu.sync_copy(src.at[tid], cidx_v)
    pltpu.sync_copy(dst.at[tid], dst_v)

    for p in range(FPC):
      if p == 0:
        base = c * FPC

        def _mkidx(j, carry):
          sl = pl.ds(j * LANES, LANES)
          cidx_v[sl] = cidx_v[sl] * 8 + base
          return carry
      else:

        def _mkidx(j, carry):
          sl = pl.ds(j * LANES, LANES)
          cidx_v[sl] = cidx_v[sl] + 1
          return carry

      lax.fori_loop(0, EPT // LANES, _mkidx, 0)

      pltpu.sync_copy(zeros, acc0_s.at[pl.ds(row0, STRIPE)])
      pltpu.sync_copy(zeros, acc1_s.at[pl.ds(row0, STRIPE)])
      plsc.subcore_barrier()

      def _window(i, carry):
        idx = cidx_v.at[pl.ds(i * KCH, KCH)]
        pltpu.async_copy(xt.at[idx], rows_v, sem).wait()
        drow = dst_v.at[i]
        pltpu.sync_copy(rows_v, acc0_s.at[drow], add=True)

        def _relu(k, carry2):
          for u in range(8):
            r = rows_v[k * 8 + u, :]
            rows_v[k * 8 + u, :] = jnp.maximum(r, 0.0)
          return carry2

        lax.fori_loop(0, KCH // 8, _relu, 0)
        pltpu.sync_copy(rows_v, acc1_s.at[drow], add=True)
        return carry

      lax.fori_loop(0, NCH, _window, 0)
      plsc.subcore_barrier()

      col = (c * FPC + p) * LANES
      pltpu.sync_copy(acc0_s.at[pl.ds(row0, STRIPE)],
                      a0.at[pl.ds(row0, STRIPE), pl.ds(col, LANES)])
      pltpu.sync_copy(acc1_s.at[pl.ds(row0, STRIPE)],
                      a1.at[pl.ds(row0, STRIPE), pl.ds(col, LANES)])
      plsc.subcore_barrier()


def _sc_aggregate(xts, srcs, dsts):
  agg_shape = jax.ShapeDtypeStruct((NP, D), jnp.float32)
  mesh = plsc.VectorSubcoreMesh(core_axis_name="c", subcore_axis_name="s")
  call = pl.kernel(
      _sc_body,
      out_type=(agg_shape,) * 6,
      mesh=mesh,
      scratch_types=[
          pltpu.VMEM((EPT,), jnp.int32),
          pltpu.VMEM((NCH, KCH), jnp.int32),
          pltpu.VMEM((KCH, LANES), jnp.float32),
          pltpu.VMEM_SHARED((NP, LANES), jnp.float32),
          pltpu.VMEM_SHARED((NP, LANES), jnp.float32),
          pltpu.SemaphoreType.DMA,
      ],
      compiler_params=pltpu.CompilerParams(use_tc_tiling_on_sc=False),
      name="hetero_gnn_segsum_sc",
  )
  zeros = jnp.zeros((STRIPE, LANES), jnp.float32)
  return call(xts[0], srcs[0], dsts[0], xts[1], srcs[1], dsts[1],
              xts[2], srcs[2], dsts[2], zeros)


def _dense_body(a00, a01, a02, a10, a11, a12, xt,
                wr00, wr01, wr02, wr10, wr11, wr12,
                wo00, wo01, wo02, wo10, wo11, wo12,
                b0, b1, mw, mb, lw, lb, out):
  dot = functools.partial(jnp.dot, preferred_element_type=jnp.float32)
  wo0 = wo00[...] + wo01[...] + wo02[...]
  wo1 = wo10[...] + wo11[...] + wo12[...]
  acc0 = (dot(a00[...], wr00[...]) + dot(a01[...], wr01[...])
          + dot(a02[...], wr02[...]) + dot(xt[...], wo0) + b0[...])
  x1 = jnp.maximum(acc0, 0.0)
  acc1 = (dot(a10[...], wr10[...]) + dot(a11[...], wr11[...])
          + dot(a12[...], wr12[...]) + dot(x1, wo1) + b1[...])
  h = dot(jnp.maximum(acc1, 0.0), mw[...]) + mb[...]
  out[...] = dot(h, lw[...]) + lb[...]


def _dense(aggs, x_target, wrs, wos, bsum0, bsum1, mlp_w, mlp_b, lin_w,
           lin_b):
  row_spec = pl.BlockSpec((BN, D), lambda i: (i, 0))
  w_spec = pl.BlockSpec((D, H), lambda i: (0, 0))
  b_spec = pl.BlockSpec((1, H), lambda i: (0, 0))
  grid = (N // BN,)
  return pl.pallas_call(
      _dense_body,
      grid=grid,
      in_specs=[row_spec] * 7 + [w_spec] * 12
      + [b_spec, b_spec, w_spec, b_spec,
         pl.BlockSpec((H, OUT), lambda i: (0, 0)),
         pl.BlockSpec((1, OUT), lambda i: (0, 0))],
      out_specs=pl.BlockSpec((BN, OUT), lambda i: (i, 0)),
      out_shape=jax.ShapeDtypeStruct((N, OUT), jnp.float32),
      name="hetero_gnn_dense_tc",
  )(*aggs, x_target, *wrs, *wos, bsum0, bsum1, mlp_w, mlp_b, lin_w, lin_b)


def _prep_edges(ei):
  npad = E_PAD - E
  pad_lane = jnp.arange(npad, dtype=jnp.int32) % LANES
  src = jnp.concatenate([ei[0], pad_lane])
  dst = jnp.concatenate([ei[1], N + pad_lane])
  return src.reshape(NTILE, EPT), dst.reshape(NTILE, NCH, KCH)


def _prep_x(x):
  return x.reshape(FCHUNK * N, LANES)


def kernel(x_app_user, x_non_app_user, x_observation, x_target,
           ei_app, ei_non, ei_obs,
           Wrel_0_0, brel_0_0, Wroot_0_0, Wrel_0_1, brel_0_1, Wroot_0_1,
           Wrel_0_2, brel_0_2, Wroot_0_2, Wrel_1_0, brel_1_0, Wroot_1_0,
           Wrel_1_1, brel_1_1, Wroot_1_1, Wrel_1_2, brel_1_2, Wroot_1_2,
           mlp_W, mlp_b, lin_W, lin_b):
  xts = [_prep_x(x) for x in (x_app_user, x_non_app_user, x_observation)]
  srcs, dsts = zip(*(_prep_edges(ei) for ei in (ei_app, ei_non, ei_obs)))

  a00, a10, a01, a11, a02, a12 = _sc_aggregate(xts, srcs, dsts)

  b0 = (brel_0_0 + brel_0_1 + brel_0_2).reshape(1, H)
  b1 = (brel_1_0 + brel_1_1 + brel_1_2).reshape(1, H)
  return _dense(
      (a00, a01, a02, a10, a11, a12), x_target,
      (Wrel_0_0, Wrel_0_1, Wrel_0_2, Wrel_1_0, Wrel_1_1, Wrel_1_2),
      (Wroot_0_0, Wroot_0_1, Wroot_0_2, Wroot_1_0, Wroot_1_1, Wroot_1_2),
      b0, b1, mlp_W, mlp_b.reshape(1, H), lin_W, lin_b.reshape(1, OUT))
